# baseline (speedup 1.0000x reference)
import functools

import jax
import jax.numpy as jnp
from jax import lax
from jax.experimental import pallas as pl
from jax.experimental.pallas import tpu as pltpu
from jax.experimental.pallas import tpu_sc as plsc

EMBED_DIM = 16
NC, NS = 2, 16
NW = NC * NS
CHUNK = 2560
NSLOT = 3


def _emb_lookup(table, ids):
    nb = ids.shape[0]
    ids_per_w = nb // NW
    n_chunks = ids_per_w // CHUNK
    mesh = plsc.VectorSubcoreMesh(core_axis_name="c", subcore_axis_name="s")

    @functools.partial(
        pl.kernel,
        mesh=mesh,
        compiler_params=pltpu.CompilerParams(use_tc_tiling_on_sc=False),
        out_type=jax.ShapeDtypeStruct((nb, EMBED_DIM), jnp.float32),
        scratch_types=[
            pltpu.VMEM((NSLOT, CHUNK), jnp.int32),
            pltpu.VMEM((NSLOT, CHUNK, EMBED_DIM), jnp.float32),
            pltpu.SemaphoreType.DMA((NSLOT,)),
            pltpu.SemaphoreType.DMA((NSLOT,)),
            pltpu.SemaphoreType.DMA((NSLOT,)),
        ],
    )
    def emb_kernel(table_hbm, idx_hbm, out_hbm, idx_v, rows_v, isem, gsem, wsem):
        wid = lax.axis_index("s") * NC + lax.axis_index("c")
        base = wid * ids_per_w

        def idx_load(g, slot):
            pltpu.async_copy(
                idx_hbm.at[pl.ds(base + g * CHUNK, CHUNK)],
                idx_v.at[slot],
                isem.at[slot],
            )

        def wait_idx(g, slot):
            pltpu.make_async_copy(
                idx_hbm.at[pl.ds(base + g * CHUNK, CHUNK)],
                idx_v.at[slot],
                isem.at[slot],
            ).wait()

        def fire_gather(slot):
            pltpu.async_copy(
                table_hbm.at[idx_v.at[slot]],
                rows_v.at[slot],
                gsem.at[slot],
            )

        def drain_gather(g, slot):
            pltpu.make_async_copy(
                out_hbm.at[pl.ds(base + g * CHUNK, CHUNK)],
                rows_v.at[slot],
                gsem.at[slot],
            ).wait()

        def writeback(g, slot):
            pltpu.async_copy(
                rows_v.at[slot],
                out_hbm.at[pl.ds(base + g * CHUNK, CHUNK)],
                wsem.at[slot],
            )

        def wait_writeback(g, slot):
            pltpu.make_async_copy(
                rows_v.at[slot],
                out_hbm.at[pl.ds(base + g * CHUNK, CHUNK)],
                wsem.at[slot],
            ).wait()

        idx_load(0, 0)
        idx_load(1, 1)
        idx_load(2, 2)
        wait_idx(0, 0)
        fire_gather(0)

        def body(g, carry):
            p = lax.rem(g, NSLOT)
            q = lax.rem(g - 1, NSLOT)
            drain_gather(g - 1, q)
            writeback(g - 1, q)
            @pl.when(g + 2 < n_chunks)
            def _():
                idx_load(g + 2, q)

            wait_idx(g, p)
            @pl.when(g >= NSLOT)
            def _():
                wait_writeback(g - NSLOT, p)

            fire_gather(p)
            return carry

        lax.fori_loop(1, n_chunks, body, 0)

        last = n_chunks - 1
        drain_gather(last, last % NSLOT)
        writeback(last, last % NSLOT)
        wait_writeback(last - 2, (last - 2) % NSLOT)
        wait_writeback(last - 1, (last - 1) % NSLOT)
        wait_writeback(last, last % NSLOT)

    return emb_kernel(table, ids)


def kernel(input_ids, table):
    b, s = input_ids.shape
    ids = input_ids.reshape(b * s).astype(jnp.int32)
    out = _emb_lookup(table, ids)
    return out.reshape(b, s, EMBED_DIM)


# speedup vs baseline: 1.0116x; 1.0116x over previous
import functools

import jax
import jax.numpy as jnp
from jax import lax
from jax.experimental import pallas as pl
from jax.experimental.pallas import tpu as pltpu
from jax.experimental.pallas import tpu_sc as plsc

EMBED_DIM = 16
NC, NS = 2, 16
NW = NC * NS
CHUNK = 3200


def _emb_lookup(table, ids):
    nb = ids.shape[0]
    ids_per_w = nb // NW
    n_chunks = ids_per_w // CHUNK
    mesh = plsc.VectorSubcoreMesh(core_axis_name="c", subcore_axis_name="s")

    @functools.partial(
        pl.kernel,
        mesh=mesh,
        compiler_params=pltpu.CompilerParams(use_tc_tiling_on_sc=False),
        out_type=jax.ShapeDtypeStruct((nb, EMBED_DIM), jnp.float32),
        scratch_types=[
            pltpu.VMEM((2, CHUNK), jnp.int32),
            pltpu.VMEM((2, CHUNK, EMBED_DIM), jnp.float32),
            pltpu.SemaphoreType.DMA((2,)),
            pltpu.SemaphoreType.DMA((2,)),
            pltpu.SemaphoreType.DMA((2,)),
        ],
    )
    def emb_kernel(table_hbm, idx_hbm, out_hbm, idx_v, rows_v, isem, gsem, wsem):
        wid = lax.axis_index("s") * NC + lax.axis_index("c")
        base = wid * ids_per_w

        def idx_load(g, slot):
            pltpu.async_copy(
                idx_hbm.at[pl.ds(base + g * CHUNK, CHUNK)],
                idx_v.at[slot],
                isem.at[slot],
            )

        def wait_idx(g, slot):
            pltpu.make_async_copy(
                idx_hbm.at[pl.ds(base + g * CHUNK, CHUNK)],
                idx_v.at[slot],
                isem.at[slot],
            ).wait()

        def fire_gather(slot):
            pltpu.async_copy(
                table_hbm.at[idx_v.at[slot]],
                rows_v.at[slot],
                gsem.at[slot],
            )

        def drain_gather(g, slot):
            pltpu.make_async_copy(
                out_hbm.at[pl.ds(base + g * CHUNK, CHUNK)],
                rows_v.at[slot],
                gsem.at[slot],
            ).wait()

        def writeback(g, slot):
            pltpu.async_copy(
                rows_v.at[slot],
                out_hbm.at[pl.ds(base + g * CHUNK, CHUNK)],
                wsem.at[slot],
            )

        def wait_writeback(g, slot):
            pltpu.make_async_copy(
                rows_v.at[slot],
                out_hbm.at[pl.ds(base + g * CHUNK, CHUNK)],
                wsem.at[slot],
            ).wait()

        idx_load(0, 0)
        idx_load(1, 1)
        wait_idx(0, 0)
        fire_gather(0)

        def body(g, carry):
            p = lax.rem(g, 2)
            q = 1 - p
            wait_idx(g, p)
            @pl.when(g >= 2)
            def _():
                wait_writeback(g - 2, p)

            fire_gather(p)
            drain_gather(g - 1, q)
            writeback(g - 1, q)
            @pl.when(g + 1 < n_chunks)
            def _():
                idx_load(g + 1, q)

            return carry

        lax.fori_loop(1, n_chunks, body, 0)

        last = n_chunks - 1
        lp = last % 2
        drain_gather(last, lp)
        writeback(last, lp)
        wait_writeback(last - 1, 1 - lp)
        wait_writeback(last, lp)

    return emb_kernel(table, ids)


def kernel(input_ids, table):
    b, s = input_ids.shape
    ids = input_ids.reshape(b * s).astype(jnp.int32)
    out = _emb_lookup(table, ids)
    return out.reshape(b, s, EMBED_DIM)


# speedup vs baseline: 1.0123x; 1.0007x over previous
import functools

import jax
import jax.numpy as jnp
from jax import lax
from jax.experimental import pallas as pl
from jax.experimental.pallas import tpu as pltpu
from jax.experimental.pallas import tpu_sc as plsc

EMBED_DIM = 16
NC, NS = 2, 16
NW = NC * NS
CHUNK = 3200


def _emb_lookup(table, ids):
    nb = ids.shape[0]
    ids_per_w = nb // NW
    n_chunks = ids_per_w // CHUNK
    assert n_chunks % 2 == 0 and n_chunks >= 4
    mesh = plsc.VectorSubcoreMesh(core_axis_name="c", subcore_axis_name="s")

    @functools.partial(
        pl.kernel,
        mesh=mesh,
        compiler_params=pltpu.CompilerParams(use_tc_tiling_on_sc=False),
        out_type=jax.ShapeDtypeStruct((nb, EMBED_DIM), jnp.float32),
        scratch_types=[
            pltpu.VMEM((CHUNK,), jnp.int32),
            pltpu.VMEM((CHUNK,), jnp.int32),
            pltpu.VMEM((CHUNK, EMBED_DIM), jnp.float32),
            pltpu.VMEM((CHUNK, EMBED_DIM), jnp.float32),
            pltpu.SemaphoreType.DMA,
            pltpu.SemaphoreType.DMA,
            pltpu.SemaphoreType.DMA,
            pltpu.SemaphoreType.DMA,
            pltpu.SemaphoreType.DMA,
            pltpu.SemaphoreType.DMA,
        ],
    )
    def emb_kernel(
        table_hbm, idx_hbm, out_hbm,
        idx_v0, idx_v1, rows_v0, rows_v1,
        isem0, isem1, gsem0, gsem1, wsem0, wsem1,
    ):
        wid = lax.axis_index("s") * NC + lax.axis_index("c")
        base = wid * ids_per_w
        slots = (
            (idx_v0, rows_v0, isem0, gsem0, wsem0),
            (idx_v1, rows_v1, isem1, gsem1, wsem1),
        )

        def out_at(g):
            return out_hbm.at[pl.ds(base + g * CHUNK, CHUNK)]

        def idx_at(g):
            return idx_hbm.at[pl.ds(base + g * CHUNK, CHUNK)]

        def idx_load(g, s):
            pltpu.async_copy(idx_at(g), s[0], s[2])

        def wait_idx(g, s):
            pltpu.make_async_copy(idx_at(g), s[0], s[2]).wait()

        def fire_gather(s):
            pltpu.async_copy(table_hbm.at[s[0]], s[1], s[3])

        def drain_gather(g, s):
            pltpu.make_async_copy(out_at(g), s[1], s[3]).wait()

        def writeback(g, s):
            pltpu.async_copy(s[1], out_at(g), s[4])

        def wait_writeback(g, s):
            pltpu.make_async_copy(s[1], out_at(g), s[4]).wait()

        def make_step(parity):
            s = slots[parity]
            o = slots[1 - parity]

            def run(g, first=False, prefetch=True):
                wait_idx(g, s)
                if not first:
                    wait_writeback(g - 2, s)
                fire_gather(s)
                drain_gather(g - 1, o)
                writeback(g - 1, o)
                if prefetch:
                    idx_load(g + 1, o)

            return run

        step1 = make_step(1)
        step0 = make_step(0)

        idx_load(0, slots[0])
        idx_load(1, slots[1])
        wait_idx(0, slots[0])
        fire_gather(slots[0])
        step1(1, first=True)
        step0(2)

        def body(t, carry):
            g = 3 + 2 * t
            step1(g)
            step0(g + 1)
            return carry

        lax.fori_loop(0, (n_chunks - 4) // 2, body, 0)

        step1(n_chunks - 1, prefetch=False)

        last = n_chunks - 1
        drain_gather(last, slots[1])
        writeback(last, slots[1])
        wait_writeback(last - 1, slots[0])
        wait_writeback(last, slots[1])

    return emb_kernel(table, ids)


def kernel(input_ids, table):
    b, s = input_ids.shape
    ids = input_ids.reshape(b * s).astype(jnp.int32)
    out = _emb_lookup(table, ids)
    return out.reshape(b, s, EMBED_DIM)
